# baseline (speedup 1.0000x reference)
import functools

import jax
import jax.numpy as jnp
from jax import lax
from jax.experimental import pallas as pl
from jax.experimental.pallas import tpu as pltpu
from jax.experimental.pallas import tpu_sc as plsc

N0, N1, N2, B = 100000, 25000, 5000, 64
E1, E2 = 500000, 500000
F_IN, H, OUT = 128, 32, 32

NC, NS = 2, 16
NW = NC * NS
K = 128

NTRASH = 16

ZR1 = 1664
ACC1 = NS * ZR1
ZR2 = 384
ACC2 = NS * ZR2


def _ceil_div(a, b):
    return -(-a // b)


def _make_sc_scatter(n_chunks, acc_rows, zr, dtype=jnp.float32,
                     spmem_table_rows=None):
    mesh = plsc.VectorSubcoreMesh(core_axis_name="c", subcore_axis_name="s")

    scratch = [
        pltpu.VMEM((n_chunks, K), jnp.int32),
        pltpu.VMEM((n_chunks, K), jnp.int32),
        pltpu.VMEM((K, H), dtype),
        pltpu.VMEM_SHARED((acc_rows, H), dtype),
        pltpu.SemaphoreType.DMA,
    ]
    if spmem_table_rows is not None:
        scratch.append(pltpu.VMEM_SHARED((spmem_table_rows, H), dtype))
        span = spmem_table_rows // NS

    @functools.partial(
        pl.kernel,
        out_type=jax.ShapeDtypeStruct((NC, acc_rows, H), dtype),
        mesh=mesh,
        compiler_params=pltpu.CompilerParams(use_tc_tiling_on_sc=False),
        scratch_types=scratch,
    )
    def sc_kernel(table_hbm, src_hbm, dst_hbm, out_hbm,
                  src_v, dst_v, rows_v, acc_sh, sem, *maybe_tbl):
        c = lax.axis_index("c")
        s = lax.axis_index("s")
        wid = s * NC + c
        base = s * zr

        if dtype == jnp.float32:
            zero16 = jnp.zeros((16,), dtype)

            def _zero_row(i, _):
                rows_v[i, pl.ds(0, 16)] = zero16
                rows_v[i, pl.ds(16, 16)] = zero16
                return 0
        else:
            zero32 = jnp.zeros((32,), dtype)

            def _zero_row(i, _):
                rows_v[i, pl.ds(0, 32)] = zero32
                return 0

        lax.fori_loop(0, K, _zero_row, 0)

        def _zero_acc(r, _):
            pltpu.sync_copy(rows_v, acc_sh.at[pl.ds(base + r * K, K)])
            return 0

        lax.fori_loop(0, zr // K, _zero_acc, 0)

        if spmem_table_rows is None:
            table = table_hbm
        else:
            table = maybe_tbl[0]
            pltpu.sync_copy(table_hbm.at[pl.ds(s * span, span)],
                            table.at[pl.ds(s * span, span)])

        pltpu.sync_copy(src_hbm.at[wid], src_v)
        pltpu.sync_copy(dst_hbm.at[wid], dst_v)
        plsc.subcore_barrier()

        def _chunk(j, _):
            pltpu.async_copy(table.at[src_v.at[j]], rows_v, sem).wait()
            pltpu.sync_copy(rows_v, acc_sh.at[dst_v.at[j]], add=True)
            return 0

        lax.fori_loop(0, n_chunks, _chunk, 0)
        plsc.subcore_barrier()

        pltpu.sync_copy(acc_sh.at[pl.ds(base, zr)],
                        out_hbm.at[c, pl.ds(base, zr)])

    return sc_kernel


def _n_chunks(n_edges):
    return _ceil_div(n_edges, NW * K)


TBL2 = 25024

_sc_scatter1 = _make_sc_scatter(_n_chunks(E1), ACC1, ZR1, dtype=jnp.bfloat16)
_sc_scatter2 = _make_sc_scatter(_n_chunks(E2), ACC2, ZR2, spmem_table_rows=TBL2)


def _prep_edges(edge_index, n_edges, dummy_dst):
    n_chunks = _n_chunks(n_edges)
    pad = NW * n_chunks * K - n_edges
    dst, src = edge_index[0], edge_index[1]
    trash = dummy_dst + (jnp.arange(pad, dtype=jnp.int32) % NTRASH)
    srcp = jnp.concatenate([src, jnp.zeros((pad,), jnp.int32)]).reshape(NW, n_chunks, K)
    dstp = jnp.concatenate([dst, trash]).reshape(NW, n_chunks, K)
    return srcp, dstp


def _premul_kernel(x_ref, w_ref, o_ref):
    o_ref[...] = jnp.dot(x_ref[...], w_ref[...],
                         preferred_element_type=jnp.float32).astype(jnp.bfloat16)


def _premul(deg_x, w1):
    blk = 10000
    return pl.pallas_call(
        _premul_kernel,
        grid=(N0 // blk,),
        in_specs=[
            pl.BlockSpec((blk, F_IN), lambda i: (i, 0)),
            pl.BlockSpec((F_IN, H), lambda i: (0, 0)),
        ],
        out_specs=pl.BlockSpec((blk, H), lambda i: (i, 0)),
        out_shape=jax.ShapeDtypeStruct((N0, H), jnp.bfloat16),
    )(deg_x, w1)


def _norm_relu(h, s_ref, ss_ref, g_ref, be_ref, n):
    m = s_ref[...] * (1.0 / n)
    v = ss_ref[...] * (1.0 / n) - m * m
    return jnp.maximum((h - m) * lax.rsqrt(v + 1e-5) * g_ref[...] + be_ref[...], 0.0)


def _pool_block(x, batch_ref, n):
    seg = (batch_ref[...] == lax.broadcasted_iota(jnp.int32, (n, B), 1))
    return lax.dot_general(seg.astype(jnp.float32), x,
                           (((0,), (0,)), ((), ())),
                           preferred_element_type=jnp.float32)


BLK1 = 5000
_VEC = pl.BlockSpec((1, H), lambda i: (0, 0))
_ACC_SPEC = pl.BlockSpec((NC, BLK1, H), lambda i: (0, i, 0))
_ROW_SPEC = pl.BlockSpec((BLK1, H), lambda i: (i, 0))
_W_SPEC = pl.BlockSpec((H, H), lambda i: (0, 0))


def _stats1_kernel(a_ref, b1_ref, s_ref, ss_ref):
    @pl.when(pl.program_id(0) == 0)
    def _():
        s_ref[...] = jnp.zeros_like(s_ref)
        ss_ref[...] = jnp.zeros_like(ss_ref)

    h = a_ref[0] + a_ref[1] + b1_ref[...]
    s_ref[...] += jnp.sum(h, axis=0, keepdims=True)
    ss_ref[...] += jnp.sum(h * h, axis=0, keepdims=True)


def _h2_kernel(a_ref, b1_ref, s_ref, ss_ref, g1_ref, be1_ref, w2_ref, b2_ref,
               h2_ref, s2_ref, ss2_ref):
    @pl.when(pl.program_id(0) == 0)
    def _():
        s2_ref[...] = jnp.zeros_like(s2_ref)
        ss2_ref[...] = jnp.zeros_like(ss2_ref)

    h = a_ref[0] + a_ref[1] + b1_ref[...]
    x = _norm_relu(h, s_ref, ss_ref, g1_ref, be1_ref, N1)
    h2 = jnp.dot(x, w2_ref[...], preferred_element_type=jnp.float32) + b2_ref[...]
    h2_ref[...] = h2
    s2_ref[...] += jnp.sum(h2, axis=0, keepdims=True)
    ss2_ref[...] += jnp.sum(h2 * h2, axis=0, keepdims=True)


def _fin1_kernel(h2_ref, s2_ref, ss2_ref, g2_ref, be2_ref, wn_ref, batch_ref,
                 y1_ref, p1_ref):
    @pl.when(pl.program_id(0) == 0)
    def _():
        p1_ref[...] = jnp.zeros_like(p1_ref)

    x1 = _norm_relu(h2_ref[...], s2_ref, ss2_ref, g2_ref, be2_ref, N1)
    y1_ref[...] = jnp.dot(x1, wn_ref[...], preferred_element_type=jnp.float32)
    p1_ref[...] += _pool_block(x1, batch_ref, BLK1)


def _mlp2_kernel(a_ref, batch_ref, b1_ref, g1_ref, be1_ref, w2_ref, b2_ref,
                 g2_ref, be2_ref, p1_ref, linw_ref, linb_ref, o_ref):
    agg = a_ref[0, :N2, :] + a_ref[1, :N2, :]
    h = agg + b1_ref[...]
    m = jnp.mean(h, axis=0, keepdims=True)
    v = jnp.mean((h - m) * (h - m), axis=0, keepdims=True)
    x = jnp.maximum((h - m) * lax.rsqrt(v + 1e-5) * g1_ref[...] + be1_ref[...], 0.0)
    h2 = jnp.dot(x, w2_ref[...], preferred_element_type=jnp.float32) + b2_ref[...]
    m2 = jnp.mean(h2, axis=0, keepdims=True)
    v2 = jnp.mean((h2 - m2) * (h2 - m2), axis=0, keepdims=True)
    x2 = jnp.maximum((h2 - m2) * lax.rsqrt(v2 + 1e-5) * g2_ref[...] + be2_ref[...], 0.0)
    p2 = _pool_block(x2, batch_ref, N2)
    linw = linw_ref[...]
    o_ref[...] = (jnp.dot(p1_ref[...], linw[:H, :], preferred_element_type=jnp.float32)
                  + jnp.dot(p2, linw[H:, :], preferred_element_type=jnp.float32)
                  + linb_ref[...])


def _layer1_tc(a1, batch1, b1, g1, be1, w2, b2, g2, be2, wn):
    grid = (N1 // BLK1,)
    stat_shape = jax.ShapeDtypeStruct((1, H), jnp.float32)
    stat_spec = pl.BlockSpec((1, H), lambda i: (0, 0))

    s1, ss1 = pl.pallas_call(
        _stats1_kernel,
        grid=grid,
        in_specs=[_ACC_SPEC, _VEC],
        out_specs=(stat_spec, stat_spec),
        out_shape=(stat_shape, stat_shape),
    )(a1, b1)

    h2, s2, ss2 = pl.pallas_call(
        _h2_kernel,
        grid=grid,
        in_specs=[_ACC_SPEC, _VEC, _VEC, _VEC, _VEC, _VEC, _W_SPEC, _VEC],
        out_specs=(_ROW_SPEC, stat_spec, stat_spec),
        out_shape=(jax.ShapeDtypeStruct((N1, H), jnp.float32),
                   stat_shape, stat_shape),
    )(a1, b1, s1, ss1, g1, be1, w2, b2)

    y1, p1 = pl.pallas_call(
        _fin1_kernel,
        grid=grid,
        in_specs=[_ROW_SPEC, _VEC, _VEC, _VEC, _VEC, _W_SPEC,
                  pl.BlockSpec((BLK1, 1), lambda i: (i, 0))],
        out_specs=(_ROW_SPEC, pl.BlockSpec((B, H), lambda i: (0, 0))),
        out_shape=(jax.ShapeDtypeStruct((N1, H), jnp.float32),
                   jax.ShapeDtypeStruct((B, H), jnp.float32)),
    )(h2, s2, ss2, g2, be2, wn, batch1)
    return y1, p1


def kernel(deg_x, conv1_W1, conv1_b1, conv1_g1, conv1_be1, conv1_W2, conv1_b2,
           conv1_g2, conv1_be2, conv2_W1, conv2_b1, conv2_g1, conv2_be1,
           conv2_W2, conv2_b2, conv2_g2, conv2_be2, lin_W, lin_b,
           edge_index1, edge_index2, batch1, batch2):
    row = lambda v: v.reshape(1, H)

    y0 = _premul(deg_x, conv1_W1)
    src1, dst1 = _prep_edges(edge_index1, E1, N1)
    a1 = _sc_scatter1(y0, src1, dst1).astype(jnp.float32)

    y1, p1 = _layer1_tc(a1, batch1.reshape(N1, 1), row(conv1_b1),
                        row(conv1_g1), row(conv1_be1), conv1_W2,
                        row(conv1_b2), row(conv1_g2), row(conv1_be2),
                        conv2_W1)

    src2, dst2 = _prep_edges(edge_index2, E2, N2)
    a2 = _sc_scatter2(jnp.pad(y1, ((0, TBL2 - N1), (0, 0))), src2, dst2)

    out = pl.pallas_call(
        _mlp2_kernel,
        out_shape=jax.ShapeDtypeStruct((B, OUT), jnp.float32),
    )(a2, batch2.reshape(N2, 1), row(conv2_b1), row(conv2_g1), row(conv2_be1),
      conv2_W2, row(conv2_b2), row(conv2_g2), row(conv2_be2), p1, lin_W,
      lin_b.reshape(1, OUT))
    return out


# speedup vs baseline: 1.0606x; 1.0606x over previous
import functools

import jax
import jax.numpy as jnp
from jax import lax
from jax.experimental import pallas as pl
from jax.experimental.pallas import tpu as pltpu
from jax.experimental.pallas import tpu_sc as plsc

N0, N1, N2, B = 100000, 25000, 5000, 64
E1, E2 = 500000, 500000
F_IN, H, OUT = 128, 32, 32

NC, NS = 2, 16
NW = NC * NS
K = 128

NTRASH = 16

ZR1 = 1568
ACC1 = NS * ZR1
ZR2 = 384
ACC2 = NS * ZR2


def _ceil_div(a, b):
    return -(-a // b)


def _make_sc_scatter(n_chunks, acc_rows, zr, dtype=jnp.float32,
                     spmem_table_rows=None):
    mesh = plsc.VectorSubcoreMesh(core_axis_name="c", subcore_axis_name="s")

    scratch = [
        pltpu.VMEM((n_chunks, K), jnp.int32),
        pltpu.VMEM((n_chunks, K), jnp.int32),
        pltpu.VMEM((K, H), dtype),
        pltpu.VMEM_SHARED((acc_rows, H), dtype),
        pltpu.SemaphoreType.DMA,
    ]
    if spmem_table_rows is not None:
        scratch.append(pltpu.VMEM_SHARED((spmem_table_rows, H), dtype))
        span = spmem_table_rows // NS

    @functools.partial(
        pl.kernel,
        out_type=jax.ShapeDtypeStruct((NC, acc_rows, H), dtype),
        mesh=mesh,
        compiler_params=pltpu.CompilerParams(use_tc_tiling_on_sc=False),
        scratch_types=scratch,
    )
    def sc_kernel(table_hbm, src_hbm, dst_hbm, out_hbm,
                  src_v, dst_v, rows_v, acc_sh, sem, *maybe_tbl):
        c = lax.axis_index("c")
        s = lax.axis_index("s")
        wid = s * NC + c
        base = s * zr

        if dtype == jnp.float32:
            zero16 = jnp.zeros((16,), dtype)

            def _zero_row(i, _):
                rows_v[i, pl.ds(0, 16)] = zero16
                rows_v[i, pl.ds(16, 16)] = zero16
                return 0
        else:
            zero32 = jnp.zeros((32,), dtype)

            def _zero_row(i, _):
                rows_v[i, pl.ds(0, 32)] = zero32
                return 0

        lax.fori_loop(0, K, _zero_row, 0)

        def _zero_acc(r, _):
            pltpu.sync_copy(rows_v, acc_sh.at[pl.ds(base + r * K, K)])
            return 0

        lax.fori_loop(0, zr // K, _zero_acc, 0)
        if zr % K:
            pltpu.sync_copy(rows_v.at[pl.ds(0, zr % K)],
                            acc_sh.at[pl.ds(base + (zr // K) * K, zr % K)])

        if spmem_table_rows is None:
            table = table_hbm
        else:
            table = maybe_tbl[0]
            pltpu.sync_copy(table_hbm.at[pl.ds(s * span, span)],
                            table.at[pl.ds(s * span, span)])

        pltpu.sync_copy(src_hbm.at[wid], src_v)
        pltpu.sync_copy(dst_hbm.at[wid], dst_v)
        plsc.subcore_barrier()

        def _chunk(j, _):
            pltpu.async_copy(table.at[src_v.at[j]], rows_v, sem).wait()
            pltpu.sync_copy(rows_v, acc_sh.at[dst_v.at[j]], add=True)
            return 0

        lax.fori_loop(0, n_chunks, _chunk, 0)
        plsc.subcore_barrier()

        pltpu.sync_copy(acc_sh.at[pl.ds(base, zr)],
                        out_hbm.at[c, pl.ds(base, zr)])

    return sc_kernel


def _n_chunks(n_edges):
    return _ceil_div(n_edges, NW * K)


TBL2 = ACC1

_sc_scatter1 = _make_sc_scatter(_n_chunks(E1), ACC1, ZR1, dtype=jnp.bfloat16)
_sc_scatter2 = _make_sc_scatter(_n_chunks(E2), ACC2, ZR2, spmem_table_rows=TBL2)


def _prep_edges(edge_index, n_edges, dummy_dst):
    n_chunks = _n_chunks(n_edges)
    pad = NW * n_chunks * K - n_edges
    dst, src = edge_index[0], edge_index[1]
    trash = dummy_dst + (jnp.arange(pad, dtype=jnp.int32) % NTRASH)
    srcp = jnp.concatenate([src, jnp.zeros((pad,), jnp.int32)]).reshape(NW, n_chunks, K)
    dstp = jnp.concatenate([dst, trash]).reshape(NW, n_chunks, K)
    return srcp, dstp


def _premul_kernel(x_ref, w_ref, o_ref):
    o_ref[...] = jnp.dot(x_ref[...], w_ref[...],
                         preferred_element_type=jnp.float32).astype(jnp.bfloat16)


def _premul(deg_x, w1):
    blk = 10000
    return pl.pallas_call(
        _premul_kernel,
        grid=(N0 // blk,),
        in_specs=[
            pl.BlockSpec((blk, F_IN), lambda i: (i, 0)),
            pl.BlockSpec((F_IN, H), lambda i: (0, 0)),
        ],
        out_specs=pl.BlockSpec((blk, H), lambda i: (i, 0)),
        out_shape=jax.ShapeDtypeStruct((N0, H), jnp.bfloat16),
    )(deg_x, w1)


def _norm_relu(h, s_ref, ss_ref, g_ref, be_ref, n):
    m = s_ref[...] * (1.0 / n)
    v = ss_ref[...] * (1.0 / n) - m * m
    return jnp.maximum((h - m) * lax.rsqrt(v + 1e-5) * g_ref[...] + be_ref[...], 0.0)


def _pool_block(x, batch_ref, n):
    seg = (batch_ref[...] == lax.broadcasted_iota(jnp.int32, (n, B), 1))
    return lax.dot_general(seg.astype(jnp.float32), x,
                           (((0,), (0,)), ((), ())),
                           preferred_element_type=jnp.float32)


RB1 = 3136
_VEC = pl.BlockSpec((1, H), lambda i: (0, 0))
_ACC_SPEC = pl.BlockSpec((NC, RB1, H), lambda i: (0, i, 0))
_ROW_SPEC = pl.BlockSpec((RB1, H), lambda i: (i, 0))
_W_SPEC = pl.BlockSpec((H, H), lambda i: (0, 0))


def _row_mask():
    row = lax.broadcasted_iota(jnp.int32, (RB1, 1), 0) + pl.program_id(0) * RB1
    return (row < N1).astype(jnp.float32)


def _stats1_kernel(a_ref, b1_ref, s_ref, ss_ref):
    @pl.when(pl.program_id(0) == 0)
    def _():
        s_ref[...] = jnp.zeros_like(s_ref)
        ss_ref[...] = jnp.zeros_like(ss_ref)

    h = a_ref[0].astype(jnp.float32) + a_ref[1].astype(jnp.float32) + b1_ref[...]
    h = h * _row_mask()
    s_ref[...] += jnp.sum(h, axis=0, keepdims=True)
    ss_ref[...] += jnp.sum(h * h, axis=0, keepdims=True)


def _h2_kernel(a_ref, b1_ref, s_ref, ss_ref, g1_ref, be1_ref, w2_ref, b2_ref,
               h2_ref, s2_ref, ss2_ref):
    @pl.when(pl.program_id(0) == 0)
    def _():
        s2_ref[...] = jnp.zeros_like(s2_ref)
        ss2_ref[...] = jnp.zeros_like(ss2_ref)

    h = a_ref[0].astype(jnp.float32) + a_ref[1].astype(jnp.float32) + b1_ref[...]
    x = _norm_relu(h, s_ref, ss_ref, g1_ref, be1_ref, N1)
    h2 = jnp.dot(x, w2_ref[...], preferred_element_type=jnp.float32) + b2_ref[...]
    h2_ref[...] = h2
    h2m = h2 * _row_mask()
    s2_ref[...] += jnp.sum(h2m, axis=0, keepdims=True)
    ss2_ref[...] += jnp.sum(h2m * h2m, axis=0, keepdims=True)


def _fin1_kernel(h2_ref, s2_ref, ss2_ref, g2_ref, be2_ref, wn_ref, batch_ref,
                 y1_ref, p1_ref):
    @pl.when(pl.program_id(0) == 0)
    def _():
        p1_ref[...] = jnp.zeros_like(p1_ref)

    x1 = _norm_relu(h2_ref[...], s2_ref, ss2_ref, g2_ref, be2_ref, N1)
    y1_ref[...] = jnp.dot(x1, wn_ref[...], preferred_element_type=jnp.float32)
    p1_ref[...] += _pool_block(x1, batch_ref, RB1)


def _mlp2_kernel(a_ref, batch_ref, b1_ref, g1_ref, be1_ref, w2_ref, b2_ref,
                 g2_ref, be2_ref, p1_ref, linw_ref, linb_ref, o_ref):
    agg = a_ref[0, :N2, :] + a_ref[1, :N2, :]
    h = agg + b1_ref[...]
    m = jnp.mean(h, axis=0, keepdims=True)
    v = jnp.mean((h - m) * (h - m), axis=0, keepdims=True)
    x = jnp.maximum((h - m) * lax.rsqrt(v + 1e-5) * g1_ref[...] + be1_ref[...], 0.0)
    h2 = jnp.dot(x, w2_ref[...], preferred_element_type=jnp.float32) + b2_ref[...]
    m2 = jnp.mean(h2, axis=0, keepdims=True)
    v2 = jnp.mean((h2 - m2) * (h2 - m2), axis=0, keepdims=True)
    x2 = jnp.maximum((h2 - m2) * lax.rsqrt(v2 + 1e-5) * g2_ref[...] + be2_ref[...], 0.0)
    p2 = _pool_block(x2, batch_ref, N2)
    linw = linw_ref[...]
    o_ref[...] = (jnp.dot(p1_ref[...], linw[:H, :], preferred_element_type=jnp.float32)
                  + jnp.dot(p2, linw[H:, :], preferred_element_type=jnp.float32)
                  + linb_ref[...])


def _layer1_tc(a1, batch1, b1, g1, be1, w2, b2, g2, be2, wn):
    grid = (ACC1 // RB1,)
    stat_shape = jax.ShapeDtypeStruct((1, H), jnp.float32)
    stat_spec = pl.BlockSpec((1, H), lambda i: (0, 0))

    s1, ss1 = pl.pallas_call(
        _stats1_kernel,
        grid=grid,
        in_specs=[_ACC_SPEC, _VEC],
        out_specs=(stat_spec, stat_spec),
        out_shape=(stat_shape, stat_shape),
    )(a1, b1)

    h2, s2, ss2 = pl.pallas_call(
        _h2_kernel,
        grid=grid,
        in_specs=[_ACC_SPEC, _VEC, _VEC, _VEC, _VEC, _VEC, _W_SPEC, _VEC],
        out_specs=(_ROW_SPEC, stat_spec, stat_spec),
        out_shape=(jax.ShapeDtypeStruct((ACC1, H), jnp.float32),
                   stat_shape, stat_shape),
    )(a1, b1, s1, ss1, g1, be1, w2, b2)

    y1, p1 = pl.pallas_call(
        _fin1_kernel,
        grid=grid,
        in_specs=[_ROW_SPEC, _VEC, _VEC, _VEC, _VEC, _W_SPEC,
                  pl.BlockSpec((RB1, 1), lambda i: (i, 0))],
        out_specs=(_ROW_SPEC, pl.BlockSpec((B, H), lambda i: (0, 0))),
        out_shape=(jax.ShapeDtypeStruct((ACC1, H), jnp.float32),
                   jax.ShapeDtypeStruct((B, H), jnp.float32)),
    )(h2, s2, ss2, g2, be2, wn, batch1)
    return y1, p1


def kernel(deg_x, conv1_W1, conv1_b1, conv1_g1, conv1_be1, conv1_W2, conv1_b2,
           conv1_g2, conv1_be2, conv2_W1, conv2_b1, conv2_g1, conv2_be1,
           conv2_W2, conv2_b2, conv2_g2, conv2_be2, lin_W, lin_b,
           edge_index1, edge_index2, batch1, batch2):
    row = lambda v: v.reshape(1, H)

    y0 = _premul(deg_x, conv1_W1)
    src1, dst1 = _prep_edges(edge_index1, E1, N1)
    a1 = _sc_scatter1(y0, src1, dst1)

    batch1p = jnp.pad(batch1, (0, ACC1 - N1), constant_values=B).reshape(ACC1, 1)
    y1, p1 = _layer1_tc(a1, batch1p, row(conv1_b1),
                        row(conv1_g1), row(conv1_be1), conv1_W2,
                        row(conv1_b2), row(conv1_g2), row(conv1_be2),
                        conv2_W1)

    src2, dst2 = _prep_edges(edge_index2, E2, N2)
    a2 = _sc_scatter2(y1, src2, dst2)

    out = pl.pallas_call(
        _mlp2_kernel,
        out_shape=jax.ShapeDtypeStruct((B, OUT), jnp.float32),
    )(a2, batch2.reshape(N2, 1), row(conv2_b1), row(conv2_g1), row(conv2_be1),
      conv2_W2, row(conv2_b2), row(conv2_g2), row(conv2_be2), p1, lin_W,
      lin_b.reshape(1, OUT))
    return out


# speedup vs baseline: 1.1020x; 1.0391x over previous
import functools

import jax
import jax.numpy as jnp
from jax import lax
from jax.experimental import pallas as pl
from jax.experimental.pallas import tpu as pltpu
from jax.experimental.pallas import tpu_sc as plsc

N0, N1, N2, B = 100000, 25000, 5000, 64
E1, E2 = 500000, 500000
F_IN, H, OUT = 128, 32, 32

NC, NS = 2, 16
NW = NC * NS
K = 128

NTRASH = 16

ZR1 = 1568
ACC1 = NS * ZR1
ZR2 = 384
ACC2 = NS * ZR2


def _ceil_div(a, b):
    return -(-a // b)


def _make_sc_scatter(n_chunks, acc_rows, zr, dtype=jnp.float32,
                     spmem_table_rows=None):
    mesh = plsc.VectorSubcoreMesh(core_axis_name="c", subcore_axis_name="s")

    scratch = [
        pltpu.VMEM((n_chunks, K), jnp.int32),
        pltpu.VMEM((n_chunks, K), jnp.int32),
        pltpu.VMEM((2, K, H), dtype),
        pltpu.VMEM_SHARED((acc_rows, H), dtype),
        [pltpu.SemaphoreType.DMA] * 2,
        [pltpu.SemaphoreType.DMA] * 2,
    ]
    if spmem_table_rows is not None:
        scratch.append(pltpu.VMEM_SHARED((spmem_table_rows, H), dtype))
        span = spmem_table_rows // NS

    @functools.partial(
        pl.kernel,
        out_type=jax.ShapeDtypeStruct((NC, acc_rows, H), dtype),
        mesh=mesh,
        compiler_params=pltpu.CompilerParams(use_tc_tiling_on_sc=False),
        scratch_types=scratch,
    )
    def sc_kernel(table_hbm, src_hbm, dst_hbm, out_hbm,
                  src_v, dst_v, rows_v, acc_sh, gsems, ssems, *maybe_tbl):
        c = lax.axis_index("c")
        s = lax.axis_index("s")
        wid = s * NC + c
        base = s * zr

        if dtype == jnp.float32:
            zero16 = jnp.zeros((16,), dtype)

            def _zero_row(i, _):
                rows_v[0, i, pl.ds(0, 16)] = zero16
                rows_v[0, i, pl.ds(16, 16)] = zero16
                return 0
        else:
            zero32 = jnp.zeros((32,), dtype)

            def _zero_row(i, _):
                rows_v[0, i, pl.ds(0, 32)] = zero32
                return 0

        lax.fori_loop(0, K, _zero_row, 0)

        def _zero_acc(r, _):
            pltpu.sync_copy(rows_v.at[0], acc_sh.at[pl.ds(base + r * K, K)])
            return 0

        lax.fori_loop(0, zr // K, _zero_acc, 0)
        if zr % K:
            pltpu.sync_copy(rows_v.at[0, pl.ds(0, zr % K)],
                            acc_sh.at[pl.ds(base + (zr // K) * K, zr % K)])

        if spmem_table_rows is None:
            table = table_hbm
        else:
            table = maybe_tbl[0]
            pltpu.sync_copy(table_hbm.at[pl.ds(s * span, span)],
                            table.at[pl.ds(s * span, span)])

        pltpu.sync_copy(src_hbm.at[wid], src_v)
        pltpu.sync_copy(dst_hbm.at[wid], dst_v)
        plsc.subcore_barrier()

        def _gather(j, b):
            return pltpu.make_async_copy(
                table.at[src_v.at[j]], rows_v.at[b], gsems[b])

        def _scatter(j, b):
            return pltpu.make_async_copy(
                rows_v.at[b], acc_sh.at[dst_v.at[j]], ssems[b])

        npairs = n_chunks // 2
        _gather(0, 0).start()

        def _pair(t, _):
            j = t * 2
            _gather(j, 0).wait()
            _scatter(j, 0).start(add=True)
            _gather(j + 1, 1).start()
            _gather(j + 1, 1).wait()
            _scatter(j + 1, 1).start(add=True)
            _scatter(j, 0).wait()

            @pl.when(t + 1 < npairs)
            def _():
                _gather(j + 2, 0).start()

            _scatter(j + 1, 1).wait()
            return 0

        lax.fori_loop(0, npairs, _pair, 0)
        plsc.subcore_barrier()

        pltpu.sync_copy(acc_sh.at[pl.ds(base, zr)],
                        out_hbm.at[c, pl.ds(base, zr)])

    return sc_kernel


def _n_chunks(n_edges):
    return _ceil_div(_ceil_div(n_edges, NW * K), 2) * 2


TBL2 = ACC1

_sc_scatter1 = _make_sc_scatter(_n_chunks(E1), ACC1, ZR1, dtype=jnp.bfloat16)
_sc_scatter2 = _make_sc_scatter(_n_chunks(E2), ACC2, ZR2, spmem_table_rows=TBL2)


def _prep_edges(edge_index, n_edges, dummy_dst):
    n_chunks = _n_chunks(n_edges)
    pad = NW * n_chunks * K - n_edges
    dst, src = edge_index[0], edge_index[1]
    trash = dummy_dst + (jnp.arange(pad, dtype=jnp.int32) % NTRASH)
    srcp = jnp.concatenate([src, jnp.zeros((pad,), jnp.int32)]).reshape(NW, n_chunks, K)
    dstp = jnp.concatenate([dst, trash]).reshape(NW, n_chunks, K)
    return srcp, dstp


def _premul_kernel(x_ref, w_ref, o_ref):
    o_ref[...] = jnp.dot(x_ref[...], w_ref[...],
                         preferred_element_type=jnp.float32).astype(jnp.bfloat16)


def _premul(deg_x, w1):
    blk = 10000
    return pl.pallas_call(
        _premul_kernel,
        grid=(N0 // blk,),
        in_specs=[
            pl.BlockSpec((blk, F_IN), lambda i: (i, 0)),
            pl.BlockSpec((F_IN, H), lambda i: (0, 0)),
        ],
        out_specs=pl.BlockSpec((blk, H), lambda i: (i, 0)),
        out_shape=jax.ShapeDtypeStruct((N0, H), jnp.bfloat16),
    )(deg_x, w1)


def _norm_relu(h, s_ref, ss_ref, g_ref, be_ref, n):
    m = s_ref[...] * (1.0 / n)
    v = ss_ref[...] * (1.0 / n) - m * m
    return jnp.maximum((h - m) * lax.rsqrt(v + 1e-5) * g_ref[...] + be_ref[...], 0.0)


def _pool_block(x, batch_ref, n):
    seg = (batch_ref[...] == lax.broadcasted_iota(jnp.int32, (n, B), 1))
    return lax.dot_general(seg.astype(jnp.float32), x,
                           (((0,), (0,)), ((), ())),
                           preferred_element_type=jnp.float32)


RB1 = 3136
_VEC = pl.BlockSpec((1, H), lambda i: (0, 0))
_ACC_SPEC = pl.BlockSpec((NC, RB1, H), lambda i: (0, i, 0))
_ROW_SPEC = pl.BlockSpec((RB1, H), lambda i: (i, 0))
_W_SPEC = pl.BlockSpec((H, H), lambda i: (0, 0))


def _row_mask():
    row = lax.broadcasted_iota(jnp.int32, (RB1, 1), 0) + pl.program_id(0) * RB1
    return (row < N1).astype(jnp.float32)


def _stats1_kernel(a_ref, b1_ref, s_ref, ss_ref):
    @pl.when(pl.program_id(0) == 0)
    def _():
        s_ref[...] = jnp.zeros_like(s_ref)
        ss_ref[...] = jnp.zeros_like(ss_ref)

    h = a_ref[0].astype(jnp.float32) + a_ref[1].astype(jnp.float32) + b1_ref[...]
    h = h * _row_mask()
    s_ref[...] += jnp.sum(h, axis=0, keepdims=True)
    ss_ref[...] += jnp.sum(h * h, axis=0, keepdims=True)


def _h2_kernel(a_ref, b1_ref, s_ref, ss_ref, g1_ref, be1_ref, w2_ref, b2_ref,
               h2_ref, s2_ref, ss2_ref):
    @pl.when(pl.program_id(0) == 0)
    def _():
        s2_ref[...] = jnp.zeros_like(s2_ref)
        ss2_ref[...] = jnp.zeros_like(ss2_ref)

    h = a_ref[0].astype(jnp.float32) + a_ref[1].astype(jnp.float32) + b1_ref[...]
    x = _norm_relu(h, s_ref, ss_ref, g1_ref, be1_ref, N1)
    h2 = jnp.dot(x, w2_ref[...], preferred_element_type=jnp.float32) + b2_ref[...]
    h2_ref[...] = h2
    h2m = h2 * _row_mask()
    s2_ref[...] += jnp.sum(h2m, axis=0, keepdims=True)
    ss2_ref[...] += jnp.sum(h2m * h2m, axis=0, keepdims=True)


def _fin1_kernel(h2_ref, s2_ref, ss2_ref, g2_ref, be2_ref, wn_ref, batch_ref,
                 y1_ref, p1_ref):
    @pl.when(pl.program_id(0) == 0)
    def _():
        p1_ref[...] = jnp.zeros_like(p1_ref)

    x1 = _norm_relu(h2_ref[...], s2_ref, ss2_ref, g2_ref, be2_ref, N1)
    y1_ref[...] = jnp.dot(x1, wn_ref[...], preferred_element_type=jnp.float32)
    p1_ref[...] += _pool_block(x1, batch_ref, RB1)


def _mlp2_kernel(a_ref, batch_ref, b1_ref, g1_ref, be1_ref, w2_ref, b2_ref,
                 g2_ref, be2_ref, p1_ref, linw_ref, linb_ref, o_ref):
    agg = a_ref[0, :N2, :] + a_ref[1, :N2, :]
    h = agg + b1_ref[...]
    m = jnp.mean(h, axis=0, keepdims=True)
    v = jnp.mean((h - m) * (h - m), axis=0, keepdims=True)
    x = jnp.maximum((h - m) * lax.rsqrt(v + 1e-5) * g1_ref[...] + be1_ref[...], 0.0)
    h2 = jnp.dot(x, w2_ref[...], preferred_element_type=jnp.float32) + b2_ref[...]
    m2 = jnp.mean(h2, axis=0, keepdims=True)
    v2 = jnp.mean((h2 - m2) * (h2 - m2), axis=0, keepdims=True)
    x2 = jnp.maximum((h2 - m2) * lax.rsqrt(v2 + 1e-5) * g2_ref[...] + be2_ref[...], 0.0)
    p2 = _pool_block(x2, batch_ref, N2)
    linw = linw_ref[...]
    o_ref[...] = (jnp.dot(p1_ref[...], linw[:H, :], preferred_element_type=jnp.float32)
                  + jnp.dot(p2, linw[H:, :], preferred_element_type=jnp.float32)
                  + linb_ref[...])


def _layer1_tc(a1, batch1, b1, g1, be1, w2, b2, g2, be2, wn):
    grid = (ACC1 // RB1,)
    stat_shape = jax.ShapeDtypeStruct((1, H), jnp.float32)
    stat_spec = pl.BlockSpec((1, H), lambda i: (0, 0))

    s1, ss1 = pl.pallas_call(
        _stats1_kernel,
        grid=grid,
        in_specs=[_ACC_SPEC, _VEC],
        out_specs=(stat_spec, stat_spec),
        out_shape=(stat_shape, stat_shape),
    )(a1, b1)

    h2, s2, ss2 = pl.pallas_call(
        _h2_kernel,
        grid=grid,
        in_specs=[_ACC_SPEC, _VEC, _VEC, _VEC, _VEC, _VEC, _W_SPEC, _VEC],
        out_specs=(_ROW_SPEC, stat_spec, stat_spec),
        out_shape=(jax.ShapeDtypeStruct((ACC1, H), jnp.float32),
                   stat_shape, stat_shape),
    )(a1, b1, s1, ss1, g1, be1, w2, b2)

    y1, p1 = pl.pallas_call(
        _fin1_kernel,
        grid=grid,
        in_specs=[_ROW_SPEC, _VEC, _VEC, _VEC, _VEC, _W_SPEC,
                  pl.BlockSpec((RB1, 1), lambda i: (i, 0))],
        out_specs=(_ROW_SPEC, pl.BlockSpec((B, H), lambda i: (0, 0))),
        out_shape=(jax.ShapeDtypeStruct((ACC1, H), jnp.float32),
                   jax.ShapeDtypeStruct((B, H), jnp.float32)),
    )(h2, s2, ss2, g2, be2, wn, batch1)
    return y1, p1


def kernel(deg_x, conv1_W1, conv1_b1, conv1_g1, conv1_be1, conv1_W2, conv1_b2,
           conv1_g2, conv1_be2, conv2_W1, conv2_b1, conv2_g1, conv2_be1,
           conv2_W2, conv2_b2, conv2_g2, conv2_be2, lin_W, lin_b,
           edge_index1, edge_index2, batch1, batch2):
    row = lambda v: v.reshape(1, H)

    y0 = _premul(deg_x, conv1_W1)
    src1, dst1 = _prep_edges(edge_index1, E1, N1)
    a1 = _sc_scatter1(y0, src1, dst1)

    batch1p = jnp.pad(batch1, (0, ACC1 - N1), constant_values=B).reshape(ACC1, 1)
    y1, p1 = _layer1_tc(a1, batch1p, row(conv1_b1),
                        row(conv1_g1), row(conv1_be1), conv1_W2,
                        row(conv1_b2), row(conv1_g2), row(conv1_be2),
                        conv2_W1)

    src2, dst2 = _prep_edges(edge_index2, E2, N2)
    a2 = _sc_scatter2(y1, src2, dst2)

    out = pl.pallas_call(
        _mlp2_kernel,
        out_shape=jax.ShapeDtypeStruct((B, OUT), jnp.float32),
    )(a2, batch2.reshape(N2, 1), row(conv2_b1), row(conv2_g1), row(conv2_be1),
      conv2_W2, row(conv2_b2), row(conv2_g2), row(conv2_be2), p1, lin_W,
      lin_b.reshape(1, OUT))
    return out
